# baseline (speedup 1.0000x reference)
import functools

import jax
import jax.numpy as jnp
from jax import lax
from jax.experimental import pallas as pl
from jax.experimental.pallas import tpu as pltpu
from jax.experimental.pallas import tpu_sc as plsc

N = 10000
E = 320000
IN_DIM = 128
H = 256
HH = 128
OUT_DIM = 128
NUM_LAYERS = 3

NCORES = 2
NTILES = 16
CHUNK = 128
EPAD = 327680
CPT16 = EPAD // (NTILES * CHUNK)
CPT32 = EPAD // (2 * NTILES * CHUNK)
GC = 32
NG = CPT16 // GC
NACC = 10112
FLUSH = NACC // NTILES
BLK = 1000

_MESH = plsc.VectorSubcoreMesh(core_axis_name="c", subcore_axis_name="s")


def _agg_body(xsplit, srcb, dstb, zrows, out, idx_s, idx_d, rows0, rows1,
              acc, gsem):
    c = lax.axis_index("c")
    s = lax.axis_index("s")
    rows_sl = pl.ds(s * FLUSH, FLUSH)
    table = xsplit.at[c]

    def branch_loop(j, cj):
        pltpu.sync_copy(zrows, acc.at[rows_sl])
        plsc.subcore_barrier()

        def group(g, cg):
            pltpu.sync_copy(srcb.at[j, s, g], idx_s)
            pltpu.sync_copy(dstb.at[j, s, g], idx_d)
            pltpu.make_async_copy(table.at[idx_s.at[0]], rows0, gsem).start()

            def pair(t, ct):
                k = t * 2
                pltpu.make_async_copy(table.at[idx_s.at[k]], rows0,
                                      gsem).wait()
                pltpu.make_async_copy(table.at[idx_s.at[k + 1]], rows1,
                                      gsem).start()
                pltpu.sync_copy(rows0, acc.at[idx_d.at[k]], add=True)
                pltpu.make_async_copy(table.at[idx_s.at[k + 1]], rows1,
                                      gsem).wait()

                @pl.when(t < GC // 2 - 1)
                def _():
                    pltpu.make_async_copy(table.at[idx_s.at[k + 2]], rows0,
                                          gsem).start()

                pltpu.sync_copy(rows1, acc.at[idx_d.at[k + 1]], add=True)
                return ct

            lax.fori_loop(0, GC // 2, pair, 0)
            return cg

        lax.fori_loop(0, NG, group, 0)
        plsc.subcore_barrier()
        pltpu.sync_copy(acc.at[rows_sl], out.at[j, c, rows_sl])
        return cj

    lax.fori_loop(0, 3, branch_loop, 0)


_agg_call = pl.kernel(
    _agg_body,
    out_type=jax.ShapeDtypeStruct((3, NCORES, NACC, HH), jnp.float32),
    mesh=_MESH,
    scratch_types=[
        pltpu.VMEM((GC, CHUNK), jnp.int32),
        pltpu.VMEM((GC, CHUNK), jnp.int32),
        pltpu.VMEM((CHUNK, HH), jnp.float32),
        pltpu.VMEM((CHUNK, HH), jnp.float32),
        pltpu.VMEM_SHARED((NACC, HH), jnp.float32),
        pltpu.SemaphoreType.DMA,
    ],
)


def _deg_body(dstb, ones_hbm, zrows, out, idx_d, ones_v, acc):
    c = lax.axis_index("c")
    s = lax.axis_index("s")
    w = c * NTILES + s
    rows_sl = pl.ds(s * FLUSH, FLUSH)
    pltpu.sync_copy(ones_hbm, ones_v)

    def branch_loop(j, cj):
        pltpu.sync_copy(zrows, acc.at[rows_sl])
        plsc.subcore_barrier()
        pltpu.sync_copy(dstb.at[j, w], idx_d)

        def step(k, cc):
            pltpu.sync_copy(ones_v, acc.at[idx_d.at[k]], add=True)
            return cc

        lax.fori_loop(0, CPT32, step, 0)
        plsc.subcore_barrier()
        pltpu.sync_copy(acc.at[rows_sl], out.at[j, c, rows_sl])
        return cj

    lax.fori_loop(0, 3, branch_loop, 0)


_deg_call = pl.kernel(
    _deg_body,
    out_type=jax.ShapeDtypeStruct((3, NCORES, NACC, HH), jnp.float32),
    mesh=_MESH,
    scratch_types=[
        pltpu.VMEM((CPT32, CHUNK), jnp.int32),
        pltpu.VMEM((CHUNK, HH), jnp.float32),
        pltpu.VMEM_SHARED((NACC, HH), jnp.float32),
    ],
)


def _pre_body(x_ref, w_ref, b_ref, xs_ref, m_ref):
    h = jnp.dot(x_ref[...], w_ref[...],
                preferred_element_type=jnp.float32) + b_ref[...]
    xs_ref[0] = h[:, :HH]
    xs_ref[1] = h[:, HH:]
    m_ref[...] = jnp.maximum(h, 0.0)


def _pre_call(x, w, b):
    return pl.pallas_call(
        _pre_body,
        grid=(N // BLK,),
        in_specs=[
            pl.BlockSpec((BLK, IN_DIM), lambda i: (i, 0)),
            pl.BlockSpec((IN_DIM, H), lambda i: (0, 0)),
            pl.BlockSpec((1, H), lambda i: (0, 0)),
        ],
        out_specs=[
            pl.BlockSpec((2, BLK, HH), lambda i: (0, i, 0)),
            pl.BlockSpec((BLK, H), lambda i: (i, 0)),
        ],
        out_shape=[
            jax.ShapeDtypeStruct((2, N, HH), jnp.float32),
            jax.ShapeDtypeStruct((N, H), jnp.float32),
        ],
    )(x, w, b)


def _layer_body(xs, ag, dg, ws, wn, w2, b2, m_in, xs_o, m_o):
    xc = jnp.concatenate([xs[0], xs[1]], axis=1)
    tot = b2[...]
    for j in range(3):
        inv = 1.0 / jnp.maximum(dg[j, 0, :, 0:1] + dg[j, 1, :, 0:1], 1.0)
        mean = jnp.concatenate([ag[j, 0], ag[j, 1]], axis=1) * inv
        o = jnp.maximum(
            jnp.dot(xc, ws[j], preferred_element_type=jnp.float32)
            + jnp.dot(mean, wn[j], preferred_element_type=jnp.float32), 0.0)
        tot = tot + jnp.dot(o, w2[j * H:(j + 1) * H],
                            preferred_element_type=jnp.float32)
    xn = jnp.maximum(tot, 0.0)
    xs_o[0] = xn[:, :HH]
    xs_o[1] = xn[:, HH:]
    m_o[...] = jnp.maximum(m_in[...], xn)


def _layer_call(xs, aggs, degs, ws, wn, w2, b2, m):
    full = lambda shape: pl.BlockSpec(shape, lambda i: tuple(0 for _ in shape))
    return pl.pallas_call(
        _layer_body,
        grid=(N // BLK,),
        in_specs=[
            pl.BlockSpec((2, BLK, HH), lambda i: (0, i, 0)),
            pl.BlockSpec((3, 2, BLK, HH), lambda i: (0, 0, i, 0)),
            pl.BlockSpec((3, 2, BLK, HH), lambda i: (0, 0, i, 0)),
            full((3, H, H)),
            full((3, H, H)),
            full((3 * H, H)),
            full((1, H)),
            pl.BlockSpec((BLK, H), lambda i: (i, 0)),
        ],
        out_specs=[
            pl.BlockSpec((2, BLK, HH), lambda i: (0, i, 0)),
            pl.BlockSpec((BLK, H), lambda i: (i, 0)),
        ],
        out_shape=[
            jax.ShapeDtypeStruct((2, N, HH), jnp.float32),
            jax.ShapeDtypeStruct((N, H), jnp.float32),
        ],
    )(xs, aggs, degs, ws, wn, w2, b2, m)


def _cls_body(m_ref, w1, b1, w2, b2, out_ref):
    hc = jnp.maximum(
        jnp.dot(m_ref[...], w1[...], preferred_element_type=jnp.float32)
        + b1[...], 0.0)
    out_ref[...] = jnp.dot(hc, w2[...],
                           preferred_element_type=jnp.float32) + b2[...]


def _cls_call(m, w1, b1, w2, b2):
    return pl.pallas_call(
        _cls_body,
        grid=(N // BLK,),
        in_specs=[
            pl.BlockSpec((BLK, H), lambda i: (i, 0)),
            pl.BlockSpec((H, H), lambda i: (0, 0)),
            pl.BlockSpec((1, H), lambda i: (0, 0)),
            pl.BlockSpec((H, OUT_DIM), lambda i: (0, 0)),
            pl.BlockSpec((1, OUT_DIM), lambda i: (0, 0)),
        ],
        out_specs=pl.BlockSpec((BLK, OUT_DIM), lambda i: (i, 0)),
        out_shape=jax.ShapeDtypeStruct((N, OUT_DIM), jnp.float32),
    )(m, w1, b1, w2, b2)


def kernel(x, edge_index, lin1_w, lin1_b, lin2_w, lin2_b, w_self, w_neigh,
           cls_w1, cls_b1, cls_w2, cls_b2):
    src = edge_index[:, 0, :]
    dst = edge_index[:, 1, :]
    pad = EPAD - E
    srcp = jnp.pad(src, ((0, 0), (0, pad)))
    dstp = jnp.pad(dst, ((0, 0), (0, pad)), constant_values=N)
    src16 = srcp.reshape(3, NTILES, NG, GC, CHUNK)
    dst16 = dstp.reshape(3, NTILES, NG, GC, CHUNK)
    dst32 = dstp.reshape(3, 2 * NTILES, CPT32, CHUNK)

    zrows = jnp.zeros((FLUSH, HH), jnp.float32)
    ones128 = jnp.ones((CHUNK, HH), jnp.float32)
    b1 = lin1_b.reshape(1, H)
    b2 = lin2_b.reshape(1, H)
    cb1 = cls_b1.reshape(1, H)
    cb2 = cls_b2.reshape(1, OUT_DIM)

    degs = _deg_call(dst32, ones128, zrows)

    xs, m = _pre_call(x, lin1_w, b1)
    for i in range(NUM_LAYERS):
        aggs = _agg_call(xs, src16, dst16, zrows)
        xs, m = _layer_call(xs, aggs, degs,
                            w_self[3 * i:3 * i + 3],
                            w_neigh[3 * i:3 * i + 3],
                            lin2_w, b2, m)
    logits = _cls_call(m, cls_w1, cb1, cls_w2, cb2)
    return (logits, m)


# speedup vs baseline: 1.0607x; 1.0607x over previous
import functools

import jax
import jax.numpy as jnp
from jax import lax
from jax.experimental import pallas as pl
from jax.experimental.pallas import tpu as pltpu
from jax.experimental.pallas import tpu_sc as plsc

N = 10000
E = 320000
IN_DIM = 128
H = 256
HH = 128
OUT_DIM = 128
NUM_LAYERS = 3

NCORES = 2
NTILES = 16
CHUNK = 128
EPAD = 327680
CPT16 = EPAD // (NTILES * CHUNK)
CPT32 = EPAD // (2 * NTILES * CHUNK)
GC = 32
NG = CPT16 // GC
NACC = 10112
FLUSH = NACC // NTILES
BLK = 1000

_MESH = plsc.VectorSubcoreMesh(core_axis_name="c", subcore_axis_name="s")


def _agg_body(xsplit, srcb, dstb, zrows, out, idx_s, idx_d, rows0, rows1,
              acc, gsem):
    c = lax.axis_index("c")
    s = lax.axis_index("s")
    pltpu.sync_copy(zrows, acc.at[pl.ds(s * FLUSH, FLUSH)])
    plsc.subcore_barrier()

    table = xsplit.at[c]

    def group(g, carry):
        pltpu.sync_copy(srcb.at[s, g], idx_s)
        pltpu.sync_copy(dstb.at[s, g], idx_d)
        pltpu.make_async_copy(table.at[idx_s.at[0]], rows0, gsem).start()

        def pair(t, cc):
            j = t * 2
            pltpu.make_async_copy(table.at[idx_s.at[j]], rows0, gsem).wait()
            pltpu.make_async_copy(table.at[idx_s.at[j + 1]], rows1,
                                  gsem).start()
            pltpu.sync_copy(rows0, acc.at[idx_d.at[j]], add=True)
            pltpu.make_async_copy(table.at[idx_s.at[j + 1]], rows1,
                                  gsem).wait()

            @pl.when(t < GC // 2 - 1)
            def _():
                pltpu.make_async_copy(table.at[idx_s.at[j + 2]], rows0,
                                      gsem).start()

            pltpu.sync_copy(rows1, acc.at[idx_d.at[j + 1]], add=True)
            return cc

        lax.fori_loop(0, GC // 2, pair, 0)
        return carry

    lax.fori_loop(0, NG, group, 0)
    plsc.subcore_barrier()
    r0 = s * FLUSH
    pltpu.sync_copy(acc.at[pl.ds(r0, FLUSH)], out.at[c, pl.ds(r0, FLUSH)])


_agg_call = pl.kernel(
    _agg_body,
    out_type=jax.ShapeDtypeStruct((NCORES, NACC, HH), jnp.float32),
    mesh=_MESH,
    scratch_types=[
        pltpu.VMEM((GC, CHUNK), jnp.int32),
        pltpu.VMEM((GC, CHUNK), jnp.int32),
        pltpu.VMEM((CHUNK, HH), jnp.float32),
        pltpu.VMEM((CHUNK, HH), jnp.float32),
        pltpu.VMEM_SHARED((NACC, HH), jnp.float32),
        pltpu.SemaphoreType.DMA,
    ],
)


def _deg_body(dstb, ones_hbm, zrows, out, idx_d, ones_v, acc):
    c = lax.axis_index("c")
    s = lax.axis_index("s")
    w = c * NTILES + s
    pltpu.sync_copy(ones_hbm, ones_v)
    pltpu.sync_copy(zrows, acc.at[pl.ds(s * FLUSH, FLUSH)])
    plsc.subcore_barrier()

    pltpu.sync_copy(dstb.at[w], idx_d)

    def step(k, cc):
        pltpu.sync_copy(ones_v, acc.at[idx_d.at[k]], add=True)
        return cc

    lax.fori_loop(0, CPT32, step, 0)
    plsc.subcore_barrier()
    r0 = s * FLUSH
    pltpu.sync_copy(acc.at[pl.ds(r0, FLUSH)], out.at[c, pl.ds(r0, FLUSH)])


_deg_call = pl.kernel(
    _deg_body,
    out_type=jax.ShapeDtypeStruct((NCORES, NACC, HH), jnp.float32),
    mesh=_MESH,
    scratch_types=[
        pltpu.VMEM((CPT32, CHUNK), jnp.int32),
        pltpu.VMEM((CHUNK, HH), jnp.float32),
        pltpu.VMEM_SHARED((NACC, HH), jnp.float32),
    ],
)


def _pre_body(x_ref, w_ref, b_ref, xs_ref, m_ref):
    h = jnp.dot(x_ref[...], w_ref[...],
                preferred_element_type=jnp.float32) + b_ref[...]
    xs_ref[0] = h[:, :HH]
    xs_ref[1] = h[:, HH:]
    m_ref[...] = jnp.maximum(h, 0.0)


def _pre_call(x, w, b):
    return pl.pallas_call(
        _pre_body,
        grid=(N // BLK,),
        in_specs=[
            pl.BlockSpec((BLK, IN_DIM), lambda i: (i, 0)),
            pl.BlockSpec((IN_DIM, H), lambda i: (0, 0)),
            pl.BlockSpec((1, H), lambda i: (0, 0)),
        ],
        out_specs=[
            pl.BlockSpec((2, BLK, HH), lambda i: (0, i, 0)),
            pl.BlockSpec((BLK, H), lambda i: (i, 0)),
        ],
        out_shape=[
            jax.ShapeDtypeStruct((2, N, HH), jnp.float32),
            jax.ShapeDtypeStruct((N, H), jnp.float32),
        ],
    )(x, w, b)


def _layer_body(xs, a0, a1, a2, d0, d1, d2, ws, wn, w2, b2, m_in,
                xs_o, m_o):
    xc = jnp.concatenate([xs[0], xs[1]], axis=1)
    tot = b2[...]
    for j, (ag, dg) in enumerate(((a0, d0), (a1, d1), (a2, d2))):
        inv = 1.0 / jnp.maximum(dg[0, :, 0:1] + dg[1, :, 0:1], 1.0)
        mean = jnp.concatenate([ag[0], ag[1]], axis=1) * inv
        o = jnp.maximum(
            jnp.dot(xc, ws[j], preferred_element_type=jnp.float32)
            + jnp.dot(mean, wn[j], preferred_element_type=jnp.float32), 0.0)
        tot = tot + jnp.dot(o, w2[j * H:(j + 1) * H],
                            preferred_element_type=jnp.float32)
    xn = jnp.maximum(tot, 0.0)
    xs_o[0] = xn[:, :HH]
    xs_o[1] = xn[:, HH:]
    m_o[...] = jnp.maximum(m_in[...], xn)


def _layer_call(xs, aggs, degs, ws, wn, w2, b2, m):
    full = lambda shape: pl.BlockSpec(shape, lambda i: tuple(0 for _ in shape))
    return pl.pallas_call(
        _layer_body,
        grid=(N // BLK,),
        in_specs=[
            pl.BlockSpec((2, BLK, HH), lambda i: (0, i, 0)),
            pl.BlockSpec((2, BLK, HH), lambda i: (0, i, 0)),
            pl.BlockSpec((2, BLK, HH), lambda i: (0, i, 0)),
            pl.BlockSpec((2, BLK, HH), lambda i: (0, i, 0)),
            pl.BlockSpec((2, BLK, HH), lambda i: (0, i, 0)),
            pl.BlockSpec((2, BLK, HH), lambda i: (0, i, 0)),
            pl.BlockSpec((2, BLK, HH), lambda i: (0, i, 0)),
            full((3, H, H)),
            full((3, H, H)),
            full((3 * H, H)),
            full((1, H)),
            pl.BlockSpec((BLK, H), lambda i: (i, 0)),
        ],
        out_specs=[
            pl.BlockSpec((2, BLK, HH), lambda i: (0, i, 0)),
            pl.BlockSpec((BLK, H), lambda i: (i, 0)),
        ],
        out_shape=[
            jax.ShapeDtypeStruct((2, N, HH), jnp.float32),
            jax.ShapeDtypeStruct((N, H), jnp.float32),
        ],
    )(xs, aggs[0], aggs[1], aggs[2], degs[0], degs[1], degs[2],
      ws, wn, w2, b2, m)


def _cls_body(m_ref, w1, b1, w2, b2, out_ref):
    hc = jnp.maximum(
        jnp.dot(m_ref[...], w1[...], preferred_element_type=jnp.float32)
        + b1[...], 0.0)
    out_ref[...] = jnp.dot(hc, w2[...],
                           preferred_element_type=jnp.float32) + b2[...]


def _cls_call(m, w1, b1, w2, b2):
    return pl.pallas_call(
        _cls_body,
        grid=(N // BLK,),
        in_specs=[
            pl.BlockSpec((BLK, H), lambda i: (i, 0)),
            pl.BlockSpec((H, H), lambda i: (0, 0)),
            pl.BlockSpec((1, H), lambda i: (0, 0)),
            pl.BlockSpec((H, OUT_DIM), lambda i: (0, 0)),
            pl.BlockSpec((1, OUT_DIM), lambda i: (0, 0)),
        ],
        out_specs=pl.BlockSpec((BLK, OUT_DIM), lambda i: (i, 0)),
        out_shape=jax.ShapeDtypeStruct((N, OUT_DIM), jnp.float32),
    )(m, w1, b1, w2, b2)


def kernel(x, edge_index, lin1_w, lin1_b, lin2_w, lin2_b, w_self, w_neigh,
           cls_w1, cls_b1, cls_w2, cls_b2):
    src = edge_index[:, 0, :]
    dst = edge_index[:, 1, :]
    pad = EPAD - E
    srcp = jnp.pad(src, ((0, 0), (0, pad)))
    dstp = jnp.pad(dst, ((0, 0), (0, pad)), constant_values=N)
    src16 = srcp.reshape(3, NTILES, NG, GC, CHUNK)
    dst16 = dstp.reshape(3, NTILES, NG, GC, CHUNK)

    zrows = jnp.zeros((FLUSH, HH), jnp.float32)
    b1 = lin1_b.reshape(1, H)
    b2 = lin2_b.reshape(1, H)
    cb1 = cls_b1.reshape(1, H)
    cb2 = cls_b2.reshape(1, OUT_DIM)

    dst32 = dstp.reshape(3, 2 * NTILES, CPT32, CHUNK)
    ones128 = jnp.ones((CHUNK, HH), jnp.float32)
    degs = [_deg_call(dst32[j], ones128, zrows) for j in range(3)]

    xs, m = _pre_call(x, lin1_w, b1)
    for i in range(NUM_LAYERS):
        aggs = [_agg_call(xs, src16[j], dst16[j], zrows) for j in range(3)]
        xs, m = _layer_call(xs, aggs, degs,
                            w_self[3 * i:3 * i + 3],
                            w_neigh[3 * i:3 * i + 3],
                            lin2_w, b2, m)
    logits = _cls_call(m, cls_w1, cb1, cls_w2, cb2)
    return (logits, m)
